# baseline (speedup 1.0000x reference)
import functools

import jax
import jax.numpy as jnp
from jax import lax
from jax.experimental import pallas as pl
from jax.experimental.pallas import tpu as pltpu
from jax.experimental.pallas import tpu_sc as plsc

N = 512
E = 16384
H = 4
C = 32
HC = H * C
NW = 32
CPW = HC // NW
WPH = NW // H
CH = 4096
LANES = 16
F32 = jnp.float32

_mesh = plsc.VectorSubcoreMesh(core_axis_name="c", subcore_axis_name="s")


def _splat(v):
  return jnp.full((LANES,), v, jnp.int32)


def _make_sc_gat(has_edge):
  scratch = [
      pltpu.VMEM((CPW * N,), F32),
      pltpu.VMEM((H * N,), F32),
      pltpu.VMEM((H * N,), F32),
      pltpu.VMEM((CH,), jnp.int32),
      pltpu.VMEM((CH,), jnp.int32),
      pltpu.VMEM((LANES * CPW * N,), F32),
      pltpu.VMEM((LANES * N,), F32),
      pltpu.VMEM((CPW * N,), F32),
      pltpu.VMEM((N,), F32),
  ]
  if has_edge:
    scratch.append(pltpu.VMEM((CH,), F32))

  out_type = [
      jax.ShapeDtypeStruct((HC * N,), F32),
      jax.ShapeDtypeStruct((H * N,), F32),
  ]

  @functools.partial(pl.kernel, mesh=_mesh, out_type=out_type,
                     scratch_types=scratch,
                     compiler_params=pltpu.CompilerParams(
                         needs_layout_passes=False))
  def sc_gat(*refs):
    if has_edge:
      (asrc_hbm, adst_hbm, h_hbm, src_hbm, dst_hbm, esc_hbm,
       numer_hbm, denom_hbm,
       h_v, asrc_v, adst_v, src_v, dst_v, numf, denf, nstage, dstage,
       esc_v) = refs
    else:
      (asrc_hbm, adst_hbm, h_hbm, src_hbm, dst_hbm,
       numer_hbm, denom_hbm,
       h_v, asrc_v, adst_v, src_v, dst_v, numf, denf, nstage, dstage) = refs

    wid = lax.axis_index("s") * 2 + lax.axis_index("c")
    c0 = wid * CPW
    head = wid // WPH
    owner = (wid % WPH) == 0

    pltpu.sync_copy(h_hbm.at[pl.ds(c0 * N, CPW * N)], h_v)
    pltpu.sync_copy(asrc_hbm, asrc_v)
    pltpu.sync_copy(adst_hbm, adst_v)

    def zero(ref, total):
      def zb(i, carry):
        for u in range(8):
          off = pl.multiple_of(i * 8 * LANES + u * LANES, LANES)
          ref[pl.ds(off, LANES)] = jnp.zeros((LANES,), F32)
        return carry
      lax.fori_loop(0, total // (8 * LANES), zb, 0)

    zero(numf, LANES * CPW * N)
    zero(denf, LANES * N)

    lane_n = lax.iota(jnp.int32, LANES) * (CPW * N)
    lane_d = lax.iota(jnp.int32, LANES) * N
    hbase = _splat(head * N)

    for ci in range(E // CH):
      base = ci * CH
      pltpu.sync_copy(src_hbm.at[pl.ds(base, CH)], src_v)
      pltpu.sync_copy(dst_hbm.at[pl.ds(base, CH)], dst_v)
      if has_edge:
        pltpu.sync_copy(esc_hbm.at[pl.ds(head * E + base, CH)], esc_v)

      UNROLL = 8

      def body(v, carry):
        for u in range(UNROLL):
          off = pl.multiple_of(v * UNROLL * LANES + u * LANES, LANES)
          s16 = src_v[pl.ds(off, LANES)]
          d16 = dst_v[pl.ds(off, LANES)]
          a_s = plsc.load_gather(asrc_v, [hbase + s16])
          a_d = plsc.load_gather(adst_v, [hbase + d16])
          alpha = a_s + a_d
          if has_edge:
            alpha = alpha + esc_v[pl.ds(off, LANES)]
          alpha = jnp.maximum(alpha, 0.2 * alpha)
          ex = jnp.exp(alpha)
          plsc.addupdate_scatter(denf, [lane_d + d16], ex)
          for j in range(CPW):
            hj = plsc.load_gather(h_v, [_splat(j * N) + s16])
            plsc.addupdate_scatter(numf, [lane_n + _splat(j * N) + d16],
                                   ex * hj)
        return carry

      lax.fori_loop(0, CH // (UNROLL * LANES), body, 0)

    for j in range(CPW):
      def rb(v, carry, j=j):
        off = pl.multiple_of(v * LANES, LANES)
        acc = numf[pl.ds(off + j * N, LANES)]
        for l in range(1, LANES):
          acc = acc + numf[pl.ds(off + l * CPW * N + j * N, LANES)]
        nstage[pl.ds(off + j * N, LANES)] = acc
        return carry
      lax.fori_loop(0, N // LANES, rb, 0)
    pltpu.sync_copy(nstage, numer_hbm.at[pl.ds(c0 * N, CPW * N)])

    @pl.when(owner)
    def _():
      def db(v, carry):
        off = pl.multiple_of(v * LANES, LANES)
        acc = denf[pl.ds(off, LANES)]
        for l in range(1, LANES):
          acc = acc + denf[pl.ds(off + l * N, LANES)]
        dstage[pl.ds(off, LANES)] = acc
        return carry
      lax.fori_loop(0, N // LANES, db, 0)
      pltpu.sync_copy(dstage, denom_hbm.at[pl.ds(head * N, N)])

  return sc_gat


_sc_gat_edge = _make_sc_gat(True)
_sc_gat_plain = _make_sc_gat(False)

EPW = E // NW
G = 128
GPW = EPW // G


@functools.partial(
    pl.kernel, mesh=_mesh,
    out_type=[jax.ShapeDtypeStruct((E, 64), F32),
              jax.ShapeDtypeStruct((E, 64), F32)],
    scratch_types=[
        pltpu.VMEM((GPW, G), jnp.int32),
        pltpu.VMEM((GPW, G), jnp.int32),
        pltpu.VMEM((G, 64), F32),
        pltpu.VMEM((G, 64), F32),
        pltpu.SemaphoreType.DMA,
        pltpu.SemaphoreType.DMA,
    ],
    compiler_params=pltpu.CompilerParams(
        needs_layout_passes=False, use_tc_tiling_on_sc=False))
def _sc_gather(pa_hbm, pb_hbm, src2_hbm, dst2_hbm, gs_hbm, gd_hbm,
               idxs_v, idxd_v, buf0, buf1, sem0, sem1):
  wid = lax.axis_index("s") * 2 + lax.axis_index("c")
  row0 = wid * GPW
  pltpu.sync_copy(src2_hbm.at[pl.ds(row0, GPW)], idxs_v)
  pltpu.sync_copy(dst2_hbm.at[pl.ds(row0, GPW)], idxd_v)
  tasks = ([(pa_hbm, idxs_v, j, gs_hbm) for j in range(GPW)]
           + [(pb_hbm, idxd_v, j, gd_hbm) for j in range(GPW)])
  bufs = (buf0, buf1)
  sems = (sem0, sem1)
  pending = [None, None]
  for t, (tbl, idxr, j, outh) in enumerate(tasks):
    b = t % 2
    if pending[b] is not None:
      pcp, poff, pouth = pending[b]
      pcp.wait()
      pltpu.sync_copy(bufs[b], pouth.at[pl.ds(poff, G)])
    cp = pltpu.async_copy(tbl.at[idxr.at[j]], bufs[b], sems[b])
    pending[b] = (cp, (row0 + j) * G, outh)
  for b in range(2):
    pcp, poff, pouth = pending[b]
    pcp.wait()
    pltpu.sync_copy(bufs[b], pouth.at[pl.ds(poff, G)])


def _leaky(v):
  return jnp.maximum(v, 0.2 * v)


def _tca_body(xp, w1p, m1s, m1d, we2, ate2, ef,
              h1, asrc, adst, exs, esc):
  h1_t = lax.dot_general(w1p[...], xp[...], (((1,), (1,)), ((), ())),
                         preferred_element_type=F32)
  h1[...] = h1_t
  a_s = jnp.dot(m1s[...], h1_t, preferred_element_type=F32)
  a_d = jnp.dot(m1d[...], h1_t, preferred_element_type=F32)
  asrc[...] = a_s
  adst[...] = a_d
  k2 = jnp.sum(we2[...] * ate2[...], axis=1, keepdims=True)
  esc[...] = lax.dot_general(k2, ef[...], (((1,), (1,)), ((), ())),
                             preferred_element_type=F32)
  fill = jnp.sum(ef[...]) * (1.0 / E)
  exs[...] = jnp.exp(_leaky(a_s + a_d + fill * k2))


def _combine(numer, denom, exs, h_t, bias_col):
  acc = jnp.zeros((C, N), F32)
  for h in range(H):
    num_h = numer[h * C:(h + 1) * C, :] + exs[h:h + 1, :] * h_t[h * C:(h + 1) * C, :]
    den_h = denom[h:h + 1, :] + exs[h:h + 1, :] + 1e-16
    acc = acc + num_h / den_h
  return acc * (1.0 / H) + bias_col


def _tcb_body(numer1, denom1, exs1, h1_t, b1c, w2, m2s, m2d,
              x0, h2, asrc2, adst2, exs2):
  x0_t = _combine(numer1[...], denom1[...], exs1[...], h1_t[...], b1c[...])
  x0[...] = x0_t
  h2_t = lax.dot_general(w2[...], x0_t, (((1,), (0,)), ((), ())),
                         preferred_element_type=F32)
  h2[...] = h2_t
  a_s = jnp.dot(m2s[...], h2_t, preferred_element_type=F32)
  a_d = jnp.dot(m2d[...], h2_t, preferred_element_type=F32)
  asrc2[...] = a_s
  adst2[...] = a_d
  exs2[...] = jnp.exp(_leaky(a_s + a_d))


def _tcc_body(numer2, denom2, exs2, h2_t, b2c, x0_t, fa1, fa2, fb1, fb2,
              b1row, pa, pb):
  x1_t = _combine(numer2[...], denom2[...], exs2[...], h2_t[...], b2c[...])
  x0t = x0_t[...]
  pa[...] = (lax.dot_general(x0t, fa1[...], (((0,), (1,)), ((), ())),
                             preferred_element_type=F32)
             + lax.dot_general(x1_t, fa2[...], (((0,), (1,)), ((), ())),
                               preferred_element_type=F32)
             + b1row[...])
  pb[...] = (lax.dot_general(x0t, fb1[...], (((0,), (1,)), ((), ())),
                             preferred_element_type=F32)
             + lax.dot_general(x1_t, fb2[...], (((0,), (1,)), ((), ())),
                               preferred_element_type=F32))


RB = 1024


def _tcd_body(gs, gd, w2, b2, w3, b3, b1row, out, tail):
  h1 = jnp.maximum(gs[...] + gd[...], 0.0)
  h2 = jnp.maximum(
      jnp.dot(h1, w2[...].T, preferred_element_type=F32) + b2[...], 0.0)
  lg = jnp.dot(h2, w3[...].T, preferred_element_type=F32) + b3[...]
  p = jnp.exp(lg)
  out[...] = p / jnp.sum(p, axis=1, keepdims=True)
  t1 = jnp.maximum(b1row[...], 0.0)
  t2 = jnp.maximum(
      jnp.dot(t1, w2[...].T, preferred_element_type=F32) + b2[...], 0.0)
  tl = jnp.dot(t2, w3[...].T, preferred_element_type=F32) + b3[...]
  tp = jnp.exp(tl)
  tp = tp / jnp.sum(tp, axis=1, keepdims=True)
  tail[0:1, 0:2] = tp


def _full(shape):
  return pl.BlockSpec(shape, lambda i: (0, 0))


def kernel(x, edges, edge_feature, gat1_W, gat1_We, gat1_att_src,
           gat1_att_dst, gat1_att_edge, gat1_bias, gat2_W, gat2_att_src,
           gat2_att_dst, gat2_bias, fc1_w, fc1_b, fc2_w, fc2_b, fc3_w,
           fc3_b):
  src = edges[0]
  dst = edges[1]

  xp = jnp.pad(x, ((0, 0), (0, 5)))
  w1p = jnp.pad(gat1_W, ((0, 0), (0, 5)))
  hh = jnp.arange(H)[:, None]
  kk = jnp.arange(HC)[None, :] // C
  blk = (kk == hh).astype(F32)
  m1s = blk * gat1_att_src.reshape(1, HC)
  m1d = blk * gat1_att_dst.reshape(1, HC)
  m2s = blk * gat2_att_src.reshape(1, HC)
  m2d = blk * gat2_att_dst.reshape(1, HC)
  we2 = gat1_We.reshape(H, C)
  ate2 = gat1_att_edge.reshape(H, C)
  b1c = gat1_bias.reshape(C, 1)
  b2c = gat2_bias.reshape(C, 1)
  fa1 = fc1_w[:, 0:C]
  fb1 = fc1_w[:, C:2 * C]
  fa2 = fc1_w[:, 2 * C:3 * C]
  fb2 = fc1_w[:, 3 * C:4 * C]
  b1row = fc1_b.reshape(1, 64)
  b2row = fc2_b.reshape(1, 32)
  b3row = fc3_b.reshape(1, 2)

  h1_t, asrc1, adst1, exs1, esc = pl.pallas_call(
      _tca_body,
      out_shape=[
          jax.ShapeDtypeStruct((HC, N), F32),
          jax.ShapeDtypeStruct((H, N), F32),
          jax.ShapeDtypeStruct((H, N), F32),
          jax.ShapeDtypeStruct((H, N), F32),
          jax.ShapeDtypeStruct((H, E), F32),
      ],
  )(xp, w1p, m1s, m1d, we2, ate2, edge_feature)

  numer1, denom1 = _sc_gat_edge(
      asrc1.reshape(H * N), adst1.reshape(H * N), h1_t.reshape(HC * N),
      src, dst, esc.reshape(H * E))
  numer1 = numer1.reshape(HC, N)
  denom1 = denom1.reshape(H, N)

  x0_t, h2_t, asrc2, adst2, exs2 = pl.pallas_call(
      _tcb_body,
      out_shape=[
          jax.ShapeDtypeStruct((C, N), F32),
          jax.ShapeDtypeStruct((HC, N), F32),
          jax.ShapeDtypeStruct((H, N), F32),
          jax.ShapeDtypeStruct((H, N), F32),
          jax.ShapeDtypeStruct((H, N), F32),
      ],
  )(numer1, denom1, exs1, h1_t, b1c, gat2_W, m2s, m2d)

  numer2, denom2 = _sc_gat_plain(
      asrc2.reshape(H * N), adst2.reshape(H * N), h2_t.reshape(HC * N),
      src, dst)
  numer2 = numer2.reshape(HC, N)
  denom2 = denom2.reshape(H, N)

  pa, pb = pl.pallas_call(
      _tcc_body,
      out_shape=[
          jax.ShapeDtypeStruct((N, 64), F32),
          jax.ShapeDtypeStruct((N, 64), F32),
      ],
  )(numer2, denom2, exs2, h2_t, b2c, x0_t, fa1, fa2, fb1, fb2, b1row)

  src2 = src.reshape(E // G, G)
  dst2 = dst.reshape(E // G, G)
  gs, gd = _sc_gather(pa, pb, src2, dst2)

  mlp, tail = pl.pallas_call(
      _tcd_body,
      grid=(E // RB,),
      in_specs=[
          pl.BlockSpec((RB, 64), lambda i: (i, 0)),
          pl.BlockSpec((RB, 64), lambda i: (i, 0)),
          _full(fc2_w.shape),
          _full(b2row.shape),
          _full(fc3_w.shape),
          _full(b3row.shape),
          _full(b1row.shape),
      ],
      out_specs=[
          pl.BlockSpec((RB, 2), lambda i: (i, 0)),
          pl.BlockSpec((8, 128), lambda i: (0, 0)),
      ],
      out_shape=[
          jax.ShapeDtypeStruct((E, 2), F32),
          jax.ShapeDtypeStruct((8, 128), F32),
      ],
  )(gs, gd, fc2_w, b2row, fc3_w, b3row, b1row)

  tail_row = tail[0:1, 0:2]
  return jnp.concatenate(
      [mlp, jnp.broadcast_to(tail_row, (N * N - E, 2))], axis=0)


# speedup vs baseline: 1.0068x; 1.0068x over previous
import functools

import jax
import jax.numpy as jnp
from jax import lax
from jax.experimental import pallas as pl
from jax.experimental.pallas import tpu as pltpu
from jax.experimental.pallas import tpu_sc as plsc

N = 512
E = 16384
H = 4
C = 32
HC = H * C
NW = 32
CPW = HC // NW
WPH = NW // H
CH = 4096
LANES = 16
F32 = jnp.float32

_mesh = plsc.VectorSubcoreMesh(core_axis_name="c", subcore_axis_name="s")


def _splat(v):
  return jnp.full((LANES,), v, jnp.int32)


def _make_sc_gat(has_edge):
  scratch = [
      pltpu.VMEM((CPW * N,), F32),
      pltpu.VMEM((H * N,), F32),
      pltpu.VMEM((H * N,), F32),
      pltpu.VMEM((CH,), jnp.int32),
      pltpu.VMEM((CH,), jnp.int32),
      pltpu.VMEM((LANES * CPW * N,), F32),
      pltpu.VMEM((LANES * N,), F32),
      pltpu.VMEM((CPW * N,), F32),
      pltpu.VMEM((N,), F32),
  ]
  if has_edge:
    scratch.append(pltpu.VMEM((CH,), F32))

  out_type = [
      jax.ShapeDtypeStruct((HC * N,), F32),
      jax.ShapeDtypeStruct((H * N,), F32),
  ]

  @functools.partial(pl.kernel, mesh=_mesh, out_type=out_type,
                     scratch_types=scratch,
                     compiler_params=pltpu.CompilerParams(
                         needs_layout_passes=False))
  def sc_gat(*refs):
    if has_edge:
      (asrc_hbm, adst_hbm, h_hbm, src_hbm, dst_hbm, esc_hbm,
       numer_hbm, denom_hbm,
       h_v, asrc_v, adst_v, src_v, dst_v, numf, denf, nstage, dstage,
       esc_v) = refs
    else:
      (asrc_hbm, adst_hbm, h_hbm, src_hbm, dst_hbm,
       numer_hbm, denom_hbm,
       h_v, asrc_v, adst_v, src_v, dst_v, numf, denf, nstage, dstage) = refs

    wid = lax.axis_index("s") * 2 + lax.axis_index("c")
    c0 = wid * CPW
    head = wid // WPH
    owner = (wid % WPH) == 0

    pltpu.sync_copy(h_hbm.at[pl.ds(c0 * N, CPW * N)], h_v)
    pltpu.sync_copy(asrc_hbm, asrc_v)
    pltpu.sync_copy(adst_hbm, adst_v)

    def zero(ref, total):
      def zb(i, carry):
        for u in range(8):
          off = pl.multiple_of(i * 8 * LANES + u * LANES, LANES)
          ref[pl.ds(off, LANES)] = jnp.zeros((LANES,), F32)
        return carry
      lax.fori_loop(0, total // (8 * LANES), zb, 0)

    zero(numf, LANES * CPW * N)
    zero(denf, LANES * N)

    lane_n = lax.iota(jnp.int32, LANES) * (CPW * N)
    lane_d = lax.iota(jnp.int32, LANES) * N
    hbase = _splat(head * N)

    for ci in range(E // CH):
      base = ci * CH
      pltpu.sync_copy(src_hbm.at[pl.ds(base, CH)], src_v)
      pltpu.sync_copy(dst_hbm.at[pl.ds(base, CH)], dst_v)
      if has_edge:
        pltpu.sync_copy(esc_hbm.at[pl.ds(head * E + base, CH)], esc_v)

      UNROLL = 4

      def body(v, carry):
        for u in range(UNROLL):
          off = pl.multiple_of(v * UNROLL * LANES + u * LANES, LANES)
          s16 = src_v[pl.ds(off, LANES)]
          d16 = dst_v[pl.ds(off, LANES)]
          a_s = plsc.load_gather(asrc_v, [hbase + s16])
          a_d = plsc.load_gather(adst_v, [hbase + d16])
          alpha = a_s + a_d
          if has_edge:
            alpha = alpha + esc_v[pl.ds(off, LANES)]
          alpha = jnp.maximum(alpha, 0.2 * alpha)
          ex = jnp.exp(alpha)
          plsc.addupdate_scatter(denf, [lane_d + d16], ex)
          for j in range(CPW):
            hj = plsc.load_gather(h_v, [_splat(j * N) + s16])
            plsc.addupdate_scatter(numf, [lane_n + _splat(j * N) + d16],
                                   ex * hj)
        return carry

      lax.fori_loop(0, CH // (UNROLL * LANES), body, 0)

    for j in range(CPW):
      def rb(v, carry, j=j):
        off = pl.multiple_of(v * LANES, LANES)
        acc = numf[pl.ds(off + j * N, LANES)]
        for l in range(1, LANES):
          acc = acc + numf[pl.ds(off + l * CPW * N + j * N, LANES)]
        nstage[pl.ds(off + j * N, LANES)] = acc
        return carry
      lax.fori_loop(0, N // LANES, rb, 0)
    pltpu.sync_copy(nstage, numer_hbm.at[pl.ds(c0 * N, CPW * N)])

    @pl.when(owner)
    def _():
      def db(v, carry):
        off = pl.multiple_of(v * LANES, LANES)
        acc = denf[pl.ds(off, LANES)]
        for l in range(1, LANES):
          acc = acc + denf[pl.ds(off + l * N, LANES)]
        dstage[pl.ds(off, LANES)] = acc
        return carry
      lax.fori_loop(0, N // LANES, db, 0)
      pltpu.sync_copy(dstage, denom_hbm.at[pl.ds(head * N, N)])

  return sc_gat


_sc_gat_edge = _make_sc_gat(True)
_sc_gat_plain = _make_sc_gat(False)

EPW = E // NW
G = 128
GPW = EPW // G


@functools.partial(
    pl.kernel, mesh=_mesh,
    out_type=[jax.ShapeDtypeStruct((E, 64), F32),
              jax.ShapeDtypeStruct((E, 64), F32)],
    scratch_types=[
        pltpu.VMEM((GPW, G), jnp.int32),
        pltpu.VMEM((GPW, G), jnp.int32),
        pltpu.VMEM((G, 64), F32),
        pltpu.VMEM((G, 64), F32),
        pltpu.SemaphoreType.DMA,
        pltpu.SemaphoreType.DMA,
    ],
    compiler_params=pltpu.CompilerParams(
        needs_layout_passes=False, use_tc_tiling_on_sc=False))
def _sc_gather(pa_hbm, pb_hbm, src2_hbm, dst2_hbm, gs_hbm, gd_hbm,
               idxs_v, idxd_v, buf0, buf1, sem0, sem1):
  wid = lax.axis_index("s") * 2 + lax.axis_index("c")
  row0 = wid * GPW
  pltpu.sync_copy(src2_hbm.at[pl.ds(row0, GPW)], idxs_v)
  pltpu.sync_copy(dst2_hbm.at[pl.ds(row0, GPW)], idxd_v)
  tasks = ([(pa_hbm, idxs_v, j, gs_hbm) for j in range(GPW)]
           + [(pb_hbm, idxd_v, j, gd_hbm) for j in range(GPW)])
  bufs = (buf0, buf1)
  sems = (sem0, sem1)
  pending = [None, None]
  for t, (tbl, idxr, j, outh) in enumerate(tasks):
    b = t % 2
    if pending[b] is not None:
      pcp, poff, pouth = pending[b]
      pcp.wait()
      pltpu.sync_copy(bufs[b], pouth.at[pl.ds(poff, G)])
    cp = pltpu.async_copy(tbl.at[idxr.at[j]], bufs[b], sems[b])
    pending[b] = (cp, (row0 + j) * G, outh)
  for b in range(2):
    pcp, poff, pouth = pending[b]
    pcp.wait()
    pltpu.sync_copy(bufs[b], pouth.at[pl.ds(poff, G)])


def _leaky(v):
  return jnp.maximum(v, 0.2 * v)


def _tca_body(xp, w1p, m1s, m1d, we2, ate2, ef,
              h1, asrc, adst, exs, esc):
  h1_t = lax.dot_general(w1p[...], xp[...], (((1,), (1,)), ((), ())),
                         preferred_element_type=F32)
  h1[...] = h1_t
  a_s = jnp.dot(m1s[...], h1_t, preferred_element_type=F32)
  a_d = jnp.dot(m1d[...], h1_t, preferred_element_type=F32)
  asrc[...] = a_s
  adst[...] = a_d
  k2 = jnp.sum(we2[...] * ate2[...], axis=1, keepdims=True)
  esc[...] = lax.dot_general(k2, ef[...], (((1,), (1,)), ((), ())),
                             preferred_element_type=F32)
  fill = jnp.sum(ef[...]) * (1.0 / E)
  exs[...] = jnp.exp(_leaky(a_s + a_d + fill * k2))


def _combine(numer, denom, exs, h_t, bias_col):
  acc = jnp.zeros((C, N), F32)
  for h in range(H):
    num_h = numer[h * C:(h + 1) * C, :] + exs[h:h + 1, :] * h_t[h * C:(h + 1) * C, :]
    den_h = denom[h:h + 1, :] + exs[h:h + 1, :] + 1e-16
    acc = acc + num_h / den_h
  return acc * (1.0 / H) + bias_col


def _tcb_body(numer1, denom1, exs1, h1_t, b1c, w2, m2s, m2d,
              x0, h2, asrc2, adst2, exs2):
  x0_t = _combine(numer1[...], denom1[...], exs1[...], h1_t[...], b1c[...])
  x0[...] = x0_t
  h2_t = lax.dot_general(w2[...], x0_t, (((1,), (0,)), ((), ())),
                         preferred_element_type=F32)
  h2[...] = h2_t
  a_s = jnp.dot(m2s[...], h2_t, preferred_element_type=F32)
  a_d = jnp.dot(m2d[...], h2_t, preferred_element_type=F32)
  asrc2[...] = a_s
  adst2[...] = a_d
  exs2[...] = jnp.exp(_leaky(a_s + a_d))


def _tcc_body(numer2, denom2, exs2, h2_t, b2c, x0_t, fa1, fa2, fb1, fb2,
              b1row, pa, pb):
  x1_t = _combine(numer2[...], denom2[...], exs2[...], h2_t[...], b2c[...])
  x0t = x0_t[...]
  pa[...] = (lax.dot_general(x0t, fa1[...], (((0,), (1,)), ((), ())),
                             preferred_element_type=F32)
             + lax.dot_general(x1_t, fa2[...], (((0,), (1,)), ((), ())),
                               preferred_element_type=F32)
             + b1row[...])
  pb[...] = (lax.dot_general(x0t, fb1[...], (((0,), (1,)), ((), ())),
                             preferred_element_type=F32)
             + lax.dot_general(x1_t, fb2[...], (((0,), (1,)), ((), ())),
                               preferred_element_type=F32))


RB = 1024


def _tcd_body(gs, gd, w2, b2, w3, b3, b1row, out, tail):
  h1 = jnp.maximum(gs[...] + gd[...], 0.0)
  h2 = jnp.maximum(
      jnp.dot(h1, w2[...].T, preferred_element_type=F32) + b2[...], 0.0)
  lg = jnp.dot(h2, w3[...].T, preferred_element_type=F32) + b3[...]
  p = jnp.exp(lg)
  out[...] = p / jnp.sum(p, axis=1, keepdims=True)
  t1 = jnp.maximum(b1row[...], 0.0)
  t2 = jnp.maximum(
      jnp.dot(t1, w2[...].T, preferred_element_type=F32) + b2[...], 0.0)
  tl = jnp.dot(t2, w3[...].T, preferred_element_type=F32) + b3[...]
  tp = jnp.exp(tl)
  tp = tp / jnp.sum(tp, axis=1, keepdims=True)
  tail[0:1, 0:2] = tp


def _full(shape):
  return pl.BlockSpec(shape, lambda i: (0, 0))


def kernel(x, edges, edge_feature, gat1_W, gat1_We, gat1_att_src,
           gat1_att_dst, gat1_att_edge, gat1_bias, gat2_W, gat2_att_src,
           gat2_att_dst, gat2_bias, fc1_w, fc1_b, fc2_w, fc2_b, fc3_w,
           fc3_b):
  src = edges[0]
  dst = edges[1]

  xp = jnp.pad(x, ((0, 0), (0, 5)))
  w1p = jnp.pad(gat1_W, ((0, 0), (0, 5)))
  hh = jnp.arange(H)[:, None]
  kk = jnp.arange(HC)[None, :] // C
  blk = (kk == hh).astype(F32)
  m1s = blk * gat1_att_src.reshape(1, HC)
  m1d = blk * gat1_att_dst.reshape(1, HC)
  m2s = blk * gat2_att_src.reshape(1, HC)
  m2d = blk * gat2_att_dst.reshape(1, HC)
  we2 = gat1_We.reshape(H, C)
  ate2 = gat1_att_edge.reshape(H, C)
  b1c = gat1_bias.reshape(C, 1)
  b2c = gat2_bias.reshape(C, 1)
  fa1 = fc1_w[:, 0:C]
  fb1 = fc1_w[:, C:2 * C]
  fa2 = fc1_w[:, 2 * C:3 * C]
  fb2 = fc1_w[:, 3 * C:4 * C]
  b1row = fc1_b.reshape(1, 64)
  b2row = fc2_b.reshape(1, 32)
  b3row = fc3_b.reshape(1, 2)

  h1_t, asrc1, adst1, exs1, esc = pl.pallas_call(
      _tca_body,
      out_shape=[
          jax.ShapeDtypeStruct((HC, N), F32),
          jax.ShapeDtypeStruct((H, N), F32),
          jax.ShapeDtypeStruct((H, N), F32),
          jax.ShapeDtypeStruct((H, N), F32),
          jax.ShapeDtypeStruct((H, E), F32),
      ],
  )(xp, w1p, m1s, m1d, we2, ate2, edge_feature)

  numer1, denom1 = _sc_gat_edge(
      asrc1.reshape(H * N), adst1.reshape(H * N), h1_t.reshape(HC * N),
      src, dst, esc.reshape(H * E))
  numer1 = numer1.reshape(HC, N)
  denom1 = denom1.reshape(H, N)

  x0_t, h2_t, asrc2, adst2, exs2 = pl.pallas_call(
      _tcb_body,
      out_shape=[
          jax.ShapeDtypeStruct((C, N), F32),
          jax.ShapeDtypeStruct((HC, N), F32),
          jax.ShapeDtypeStruct((H, N), F32),
          jax.ShapeDtypeStruct((H, N), F32),
          jax.ShapeDtypeStruct((H, N), F32),
      ],
  )(numer1, denom1, exs1, h1_t, b1c, gat2_W, m2s, m2d)

  numer2, denom2 = _sc_gat_plain(
      asrc2.reshape(H * N), adst2.reshape(H * N), h2_t.reshape(HC * N),
      src, dst)
  numer2 = numer2.reshape(HC, N)
  denom2 = denom2.reshape(H, N)

  pa, pb = pl.pallas_call(
      _tcc_body,
      out_shape=[
          jax.ShapeDtypeStruct((N, 64), F32),
          jax.ShapeDtypeStruct((N, 64), F32),
      ],
  )(numer2, denom2, exs2, h2_t, b2c, x0_t, fa1, fa2, fb1, fb2, b1row)

  src2 = src.reshape(E // G, G)
  dst2 = dst.reshape(E // G, G)
  gs, gd = _sc_gather(pa, pb, src2, dst2)

  mlp, tail = pl.pallas_call(
      _tcd_body,
      grid=(E // RB,),
      in_specs=[
          pl.BlockSpec((RB, 64), lambda i: (i, 0)),
          pl.BlockSpec((RB, 64), lambda i: (i, 0)),
          _full(fc2_w.shape),
          _full(b2row.shape),
          _full(fc3_w.shape),
          _full(b3row.shape),
          _full(b1row.shape),
      ],
      out_specs=[
          pl.BlockSpec((RB, 2), lambda i: (i, 0)),
          pl.BlockSpec((8, 128), lambda i: (0, 0)),
      ],
      out_shape=[
          jax.ShapeDtypeStruct((E, 2), F32),
          jax.ShapeDtypeStruct((8, 128), F32),
      ],
  )(gs, gd, fc2_w, b2row, fc3_w, b3row, b1row)

  tail_row = tail[0:1, 0:2]
  return jnp.concatenate(
      [mlp, jnp.broadcast_to(tail_row, (N * N - E, 2))], axis=0)
